# Initial kernel scaffold; baseline (speedup 1.0000x reference)
import jax
import jax.numpy as jnp
from jax.experimental import pallas as pl


def kernel(input_, weight_indices, W, A_buffer, B_buffer, DeltaW_buffer, metas_buffer, ss_buffer):
    raise NotImplementedError("write your pallas kernel here")


# speedup vs baseline: 2.0042x; 2.0042x over previous
import functools

import jax
import jax.numpy as jnp
from jax.experimental import pallas as pl
from jax.experimental.pallas import tpu as pltpu

TB = 128
CB = 1024


def _gmm_body(bex_ref, xs_ref, w_ref, a_ref, b_ref, dw_ref, metas_ref, ss_ref,
              out_ref):
    xb = xs_ref[...]
    base = jax.lax.dot_general(
        xb, w_ref[...], (((1,), (1,)), ((), ())),
        preferred_element_type=jnp.float32)
    mid = jnp.dot(xb, a_ref[0], preferred_element_type=jnp.float32)
    lora = jnp.dot(mid, b_ref[0], preferred_element_type=jnp.float32)
    dmm = jnp.dot(xb, dw_ref[0], preferred_element_type=jnp.float32)
    rs = jnp.sum(xb, axis=1, keepdims=True)
    out_ref[...] = base + lora + dmm * ss_ref[0] + rs * metas_ref[0]


def _grouped_matmul(bex, xs, W, A, B, DW, metas, ss, P):
    E, D, R2 = A.shape
    OUT = B.shape[2]
    nj = OUT // CB
    ntb = P // TB
    grid = (nj, ntb)

    def tok(j, tb, bex_ref):
        return (tb, 0)

    def wmap(j, tb, bex_ref):
        return (j, 0)

    def emap(j, tb, bex_ref):
        return (bex_ref[tb], 0, j)

    grid_spec = pltpu.PrefetchScalarGridSpec(
        num_scalar_prefetch=1,
        grid=grid,
        in_specs=[
            pl.BlockSpec((TB, D), tok),
            pl.BlockSpec((CB, D), wmap),
            pl.BlockSpec((1, D, R2), lambda j, tb, bex_ref: (bex_ref[tb], 0, 0)),
            pl.BlockSpec((1, R2, CB), emap),
            pl.BlockSpec((1, D, CB), emap),
            pl.BlockSpec((1, 1, CB), emap),
            pl.BlockSpec((1, 1, CB), emap),
        ],
        out_specs=pl.BlockSpec((TB, CB), lambda j, tb, bex_ref: (tb, j)),
    )
    return pl.pallas_call(
        _gmm_body,
        grid_spec=grid_spec,
        out_shape=jax.ShapeDtypeStruct((P, OUT), jnp.float32),
        compiler_params=pltpu.CompilerParams(
            dimension_semantics=("arbitrary", "arbitrary")),
    )(bex, xs, W, A, B, DW, metas, ss)


def kernel(input_, weight_indices, W, A_buffer, B_buffer, DeltaW_buffer,
           metas_buffer, ss_buffer):
    T, D = input_.shape
    E = A_buffer.shape[0]
    P = T + E * TB
    idx = weight_indices.astype(jnp.int32)

    onehot = jax.nn.one_hot(idx, E, dtype=jnp.int32)
    counts = jnp.sum(onehot, axis=0)
    seg = ((counts + TB - 1) // TB) * TB
    segend = jnp.cumsum(seg)
    off = segend - seg
    occ = jnp.cumsum(onehot, axis=0) - onehot
    rank = jnp.take_along_axis(occ, idx[:, None], axis=1)[:, 0]
    pos = jnp.take(off, idx) + rank
    perm = jnp.zeros((P,), jnp.int32).at[pos].set(jnp.arange(T, dtype=jnp.int32))
    bex = jnp.clip(
        jnp.searchsorted(segend, jnp.arange(P // TB, dtype=jnp.int32) * TB,
                         side="right"), 0, E - 1).astype(jnp.int32)

    xs = jnp.take(input_, perm, axis=0)

    R = A_buffer.shape[2] // 2
    bd = B_buffer.shape[2] // 2
    Bp = jnp.zeros((E, 2 * R, 2 * bd), B_buffer.dtype)
    Bp = Bp.at[:, :R, :bd].set(B_buffer[:, :, :bd])
    Bp = Bp.at[:, R:, bd:].set(B_buffer[:, :, bd:])

    out_sorted = _grouped_matmul(bex, xs, W, A_buffer, Bp, DeltaW_buffer,
                                 metas_buffer, ss_buffer, P)

    return jnp.take(out_sorted, pos, axis=0)


# speedup vs baseline: 2.2047x; 1.1000x over previous
import functools

import jax
import jax.numpy as jnp
from jax.experimental import pallas as pl
from jax.experimental.pallas import tpu as pltpu

TB = 128
CB = 1024


def _gmm_body(bex_ref, xs_ref, w_ref, a_ref, b_ref, dw_ref, metas_ref, ss_ref,
              out_ref):
    xb = xs_ref[...]
    base = jax.lax.dot_general(
        xb, w_ref[...], (((1,), (1,)), ((), ())),
        preferred_element_type=jnp.float32)
    mid = jnp.dot(xb, a_ref[0], preferred_element_type=jnp.float32)
    lora = jnp.dot(mid.astype(jnp.bfloat16), b_ref[0],
                   preferred_element_type=jnp.float32)
    dmm = jnp.dot(xb, dw_ref[0], preferred_element_type=jnp.float32)
    rs = jnp.sum(xb.astype(jnp.float32), axis=1, keepdims=True)
    out_ref[...] = base + lora + dmm * ss_ref[0] + rs * metas_ref[0]


def _grouped_matmul(bex, xs, W, A, B, DW, metas, ss, P):
    E, D, R2 = A.shape
    OUT = B.shape[2]
    nj = OUT // CB
    ntb = P // TB
    grid = (nj, ntb)

    def tok(j, tb, bex_ref):
        return (tb, 0)

    def wmap(j, tb, bex_ref):
        return (j, 0)

    def emap(j, tb, bex_ref):
        return (bex_ref[tb], 0, j)

    grid_spec = pltpu.PrefetchScalarGridSpec(
        num_scalar_prefetch=1,
        grid=grid,
        in_specs=[
            pl.BlockSpec((TB, D), tok),
            pl.BlockSpec((CB, D), wmap),
            pl.BlockSpec((1, D, R2), lambda j, tb, bex_ref: (bex_ref[tb], 0, 0)),
            pl.BlockSpec((1, R2, CB), emap),
            pl.BlockSpec((1, D, CB), emap),
            pl.BlockSpec((1, 1, CB), emap),
            pl.BlockSpec((1, 1, CB), emap),
        ],
        out_specs=pl.BlockSpec((TB, CB), lambda j, tb, bex_ref: (tb, j)),
    )
    return pl.pallas_call(
        _gmm_body,
        grid_spec=grid_spec,
        out_shape=jax.ShapeDtypeStruct((P, OUT), jnp.float32),
        compiler_params=pltpu.CompilerParams(
            dimension_semantics=("arbitrary", "arbitrary")),
    )(bex, xs, W, A, B, DW, metas, ss)


def kernel(input_, weight_indices, W, A_buffer, B_buffer, DeltaW_buffer,
           metas_buffer, ss_buffer):
    T, D = input_.shape
    E = A_buffer.shape[0]
    P = T + E * TB
    idx = weight_indices.astype(jnp.int32)

    onehot = jax.nn.one_hot(idx, E, dtype=jnp.int32)
    counts = jnp.sum(onehot, axis=0)
    seg = ((counts + TB - 1) // TB) * TB
    segend = jnp.cumsum(seg)
    off = segend - seg
    occ = jnp.cumsum(onehot, axis=0) - onehot
    rank = jnp.take_along_axis(occ, idx[:, None], axis=1)[:, 0]
    pos = jnp.take(off, idx) + rank
    perm = jnp.zeros((P,), jnp.int32).at[pos].set(jnp.arange(T, dtype=jnp.int32))
    bex = jnp.clip(
        jnp.searchsorted(segend, jnp.arange(P // TB, dtype=jnp.int32) * TB,
                         side="right"), 0, E - 1).astype(jnp.int32)

    xs = jnp.take(input_.astype(jnp.bfloat16), perm, axis=0)

    R = A_buffer.shape[2] // 2
    bd = B_buffer.shape[2] // 2
    Bp = jnp.zeros((E, 2 * R, 2 * bd), jnp.bfloat16)
    Bp = Bp.at[:, :R, :bd].set(B_buffer[:, :, :bd].astype(jnp.bfloat16))
    Bp = Bp.at[:, R:, bd:].set(B_buffer[:, :, bd:].astype(jnp.bfloat16))

    out_sorted = _grouped_matmul(
        bex, xs, W.astype(jnp.bfloat16), A_buffer.astype(jnp.bfloat16), Bp,
        DeltaW_buffer.astype(jnp.bfloat16), metas_buffer, ss_buffer, P)

    return jnp.take(out_sorted, pos, axis=0)
